# baseline (speedup 1.0000x reference)
import jax
import jax.numpy as jnp
from jax import lax
from jax.experimental import pallas as pl
from jax.experimental.pallas import tpu as pltpu

_CB = 128


def _ce2(l0, l1, label):
    m = jnp.maximum(l0, l1)
    lse = m + jnp.log1p(jnp.exp(-jnp.abs(l0 - l1)))
    picked = jnp.where(label > 0, l1, l0)
    return lse - picked


def _smooth_l1(pred, target):
    d = jnp.abs(pred - target)
    return jnp.where(d < 1.0, 0.5 * d * d, d - 0.5)


def _colsum(x):
    return jnp.sum(x, axis=0, keepdims=True)


def _row(acc_ref, i):
    return jnp.sum(acc_ref[i:i + 1, :])


def _bcast(v, w):
    return jnp.zeros((1, w), jnp.float32) + v


def _body(p_ref, trm_ref, tclm_ref, tram_ref, xm_ref, ym_ref,
          p_hbm, tr_hbm, tcl_hbm, tm_hbm, x_hbm, y_hbm,
          out_ref, acc_ref, pbuf, trbuf, tclbuf, tmbuf, xbuf, ybuf, sem):
    first = (pl.program_id(0) == 0) & (pl.program_id(1) == 0)
    last = ((pl.program_id(0) == pl.num_programs(0) - 1)
            & (pl.program_id(1) == pl.num_programs(1) - 1))

    @pl.when(first)
    def _():
        acc_ref[...] = jnp.zeros_like(acc_ref)

    pch = p_ref[0]
    tr = trm_ref[0]
    tcl = tclm_ref[0]
    tm = tram_ref[0]
    f32 = jnp.float32
    tm_f = tm.astype(f32)
    tr_f = tr.astype(f32)
    tcl_f = tcl.astype(f32)
    ce_tr = _ce2(pch[0], pch[1], tr)
    ce_tcl = _ce2(pch[2], pch[3], tcl)
    l1_x = _smooth_l1(pch[4], xm_ref[0])
    l1_y = _smooth_l1(pch[5], ym_ref[0])

    rows = [
        _colsum(tm_f),
        _colsum(tr_f * tm_f),
        _colsum(tm_f * ce_tr),
        _colsum(tcl_f * tm_f),
        _colsum(tm_f * ce_tcl),
        _colsum(tm_f * l1_x),
        _colsum(tm_f * l1_y),
        jnp.zeros((1, tm_f.shape[1]), f32),
    ]
    acc_ref[...] += jnp.concatenate(rows, axis=0)

    @pl.when(last)
    def _():
        ntm = _row(acc_ref, 0)
        npos_tr = _row(acc_ref, 1)
        num_tr = _row(acc_ref, 2)
        npos_tcl = _row(acc_ref, 3)
        num_tcl = _row(acc_ref, 4)
        num_x = _row(acc_ref, 5)
        num_y = _row(acc_ref, 6)
        w = out_ref.shape[1]
        inv = 1.0 / ntm
        out_ref[...] = jnp.concatenate([
            _bcast(num_tr * inv, w),
            _bcast(num_tcl * inv, w),
            _bcast(num_x * inv, w),
            _bcast(num_y * inv, w),
            _bcast(npos_tr, w),
            _bcast(npos_tcl, w),
            _bcast(ntm, w),
            jnp.zeros((1, w), jnp.float32),
        ], axis=0)

        nneg_tr = ntm - npos_tr
        nneg_tcl = ntm - npos_tcl
        need_rare = (3.0 * npos_tr < nneg_tr) | (3.0 * npos_tcl < nneg_tcl)

        @pl.when(need_rare)
        def _():
            B = p_hbm.shape[0]
            H = p_hbm.shape[2]
            n_h = H // _CB
            n_chunks = B * n_h
            k_tr = jnp.minimum(nneg_tr, 3.0 * npos_tr)
            k_tcl = jnp.minimum(nneg_tcl, 3.0 * npos_tcl)
            ks = (k_tr, k_tcl, k_tr, k_tr)

            def dma_chunk(tc):
                b = tc // n_h
                h0 = (tc % n_h) * _CB
                cps = []
                for c in range(6):
                    cps.append(pltpu.make_async_copy(
                        p_hbm.at[b, c, pl.ds(h0, _CB), :], pbuf.at[c], sem))
                cps.append(pltpu.make_async_copy(
                    tr_hbm.at[b, pl.ds(h0, _CB), :], trbuf, sem))
                cps.append(pltpu.make_async_copy(
                    tcl_hbm.at[b, pl.ds(h0, _CB), :], tclbuf, sem))
                cps.append(pltpu.make_async_copy(
                    tm_hbm.at[b, pl.ds(h0, _CB), :], tmbuf, sem))
                cps.append(pltpu.make_async_copy(
                    x_hbm.at[b, pl.ds(h0, _CB), :], xbuf, sem))
                cps.append(pltpu.make_async_copy(
                    y_hbm.at[b, pl.ds(h0, _CB), :], ybuf, sem))
                for cp in cps:
                    cp.start()
                for cp in cps:
                    cp.wait()

            def chunk_fields():
                ctr = trbuf[...]
                ctcl = tclbuf[...]
                ctm = tmbuf[...]
                cpos_tr = (ctr * ctm) > 0
                cneg_tr = ((1 - ctr) * ctm) > 0
                cpos_tcl = (ctcl * ctm) > 0
                cneg_tcl = ((1 - ctcl) * ctm) > 0
                cce_tr = _ce2(pbuf[0], pbuf[1], ctr)
                cce_tcl = _ce2(pbuf[2], pbuf[3], ctcl)
                cl1_x = _smooth_l1(pbuf[4], xbuf[...])
                cl1_y = _smooth_l1(pbuf[5], ybuf[...])
                return ((cce_tr, cneg_tr, cpos_tr),
                        (cce_tcl, cneg_tcl, cpos_tcl),
                        (cl1_x, cneg_tr, cpos_tr),
                        (cl1_y, cneg_tr, cpos_tr))

            def probe(thrs):
                def cbody(tc, carry):
                    dma_chunk(tc)
                    fields = chunk_fields()
                    outs = []
                    for i in range(4):
                        lossv, neg, _ = fields[i]
                        bits = lax.bitcast_convert_type(lossv, jnp.int32)
                        sel = neg & (bits >= thrs[i])
                        outs.append(carry[i] + jnp.sum(sel.astype(jnp.float32)))
                    return tuple(outs)
                return lax.fori_loop(0, n_chunks, cbody, (0.0,) * 4)

            def bit_body(i, prefix):
                shift = jnp.int32(30) - i
                cands = tuple(p | (jnp.int32(1) << shift) for p in prefix)
                ge = probe(cands)
                return tuple(
                    jnp.where(ge[j] >= ks[j], cands[j], prefix[j])
                    for j in range(4))

            prefix = lax.fori_loop(0, 31, bit_body, (jnp.int32(0),) * 4)

            def fbody(tc, carry):
                dma_chunk(tc)
                fields = chunk_fields()
                outs = list(carry)
                for i in range(4):
                    lossv, neg, pos = fields[i]
                    bits = lax.bitcast_convert_type(lossv, jnp.int32)
                    gt = neg & (bits > prefix[i])
                    outs[3 * i] += jnp.sum(gt.astype(jnp.float32))
                    outs[3 * i + 1] += jnp.sum(jnp.where(gt, lossv, 0.0))
                    outs[3 * i + 2] += jnp.sum(jnp.where(pos, lossv, 0.0))
                return tuple(outs)

            fin = lax.fori_loop(0, n_chunks, fbody, (0.0,) * 12)

            dens = (npos_tr + k_tr, npos_tcl + k_tcl,
                    npos_tr + k_tr, npos_tr + k_tr)
            for i in range(4):
                gt_cnt, gt_sum, ps = fin[3 * i], fin[3 * i + 1], fin[3 * i + 2]
                kth = lax.bitcast_convert_type(prefix[i], jnp.float32)
                top = gt_sum + (ks[i] - gt_cnt) * kth
                top = jnp.where(ks[i] == 0.0, 0.0, top)
                out_ref[i:i + 1, :] = _bcast((ps + top) / dens[i], w)


def kernel(predict, tr_mask, tcl_mask, sin_map, cos_map, radii_map,
           train_mask, x_mask, y_mask):
    del sin_map, cos_map, radii_map
    B, C, H, W = predict.shape
    Hb = 512 if H % 512 == 0 else H
    grid = (B, H // Hb)

    tr_i = tr_mask.astype(jnp.int32)
    tcl_i = tcl_mask.astype(jnp.int32)
    tm_i = train_mask.astype(jnp.int32)

    p_spec = pl.BlockSpec((1, 6, Hb, W), lambda b, h: (b, 0, h, 0))
    m_spec = pl.BlockSpec((1, Hb, W), lambda b, h: (b, h, 0))
    hbm = pl.BlockSpec(memory_space=pltpu.MemorySpace.HBM)

    out = pl.pallas_call(
        _body,
        grid=grid,
        in_specs=[p_spec, m_spec, m_spec, m_spec, m_spec, m_spec,
                  hbm, hbm, hbm, hbm, hbm, hbm],
        out_specs=pl.BlockSpec((8, 128), lambda b, h: (0, 0)),
        out_shape=jax.ShapeDtypeStruct((8, 128), jnp.float32),
        scratch_shapes=[
            pltpu.VMEM((8, W), jnp.float32),
            pltpu.VMEM((6, _CB, W), jnp.float32),
            pltpu.VMEM((_CB, W), jnp.int32),
            pltpu.VMEM((_CB, W), jnp.int32),
            pltpu.VMEM((_CB, W), jnp.int32),
            pltpu.VMEM((_CB, W), jnp.float32),
            pltpu.VMEM((_CB, W), jnp.float32),
            pltpu.SemaphoreType.DMA,
        ],
    )(predict, tr_i, tcl_i, tm_i, x_mask, y_mask,
      predict, tr_i, tcl_i, tm_i, x_mask, y_mask)

    return (out[0, 0], out[1, 0], out[2, 0] / 2, out[3, 0] / 2)


# speedup vs baseline: 1.0596x; 1.0596x over previous
import jax
import jax.numpy as jnp
from jax import lax
from jax.experimental import pallas as pl
from jax.experimental.pallas import tpu as pltpu

_CB = 128


def _ce2(l0, l1, label):
    m = jnp.maximum(l0, l1)
    lse = m + jnp.log1p(jnp.exp(-jnp.abs(l0 - l1)))
    picked = jnp.where(label > 0, l1, l0)
    return lse - picked


def _smooth_l1(pred, target):
    d = jnp.abs(pred - target)
    return jnp.where(d < 1.0, 0.5 * d * d, d - 0.5)


def _colsum(x):
    return jnp.sum(x, axis=0, keepdims=True)


def _row(acc_ref, i):
    return jnp.sum(acc_ref[i:i + 1, :])


def _bcast(v, w):
    return jnp.zeros((1, w), jnp.float32) + v


def _body(p_ref, trm_ref, tclm_ref, tram_ref, xm_ref, ym_ref,
          p_hbm, tr_hbm, tcl_hbm, tm_hbm, x_hbm, y_hbm,
          out_ref, acc_ref, pbuf, trbuf, tclbuf, tmbuf, xbuf, ybuf, sem):
    first = (pl.program_id(0) == 0) & (pl.program_id(1) == 0)
    last = ((pl.program_id(0) == pl.num_programs(0) - 1)
            & (pl.program_id(1) == pl.num_programs(1) - 1))

    @pl.when(first)
    def _():
        acc_ref[...] = jnp.zeros_like(acc_ref)

    pch = p_ref[0]
    tr = trm_ref[0]
    tcl = tclm_ref[0]
    tm = tram_ref[0]
    f32 = jnp.float32
    tm_f = tm.astype(f32)
    tr_f = tr.astype(f32)
    tcl_f = tcl.astype(f32)
    ce_tr = _ce2(pch[0], pch[1], tr)
    ce_tcl = _ce2(pch[2], pch[3], tcl)
    l1_x = _smooth_l1(pch[4], xm_ref[0])
    l1_y = _smooth_l1(pch[5], ym_ref[0])

    rows = [
        _colsum(tm_f),
        _colsum(tr_f * tm_f),
        _colsum(tm_f * ce_tr),
        _colsum(tcl_f * tm_f),
        _colsum(tm_f * ce_tcl),
        _colsum(tm_f * l1_x),
        _colsum(tm_f * l1_y),
        jnp.zeros((1, tm_f.shape[1]), f32),
    ]
    acc_ref[...] += jnp.concatenate(rows, axis=0)

    @pl.when(last)
    def _():
        ntm = _row(acc_ref, 0)
        npos_tr = _row(acc_ref, 1)
        num_tr = _row(acc_ref, 2)
        npos_tcl = _row(acc_ref, 3)
        num_tcl = _row(acc_ref, 4)
        num_x = _row(acc_ref, 5)
        num_y = _row(acc_ref, 6)
        w = out_ref.shape[1]
        inv = 1.0 / ntm
        out_ref[...] = jnp.concatenate([
            _bcast(num_tr * inv, w),
            _bcast(num_tcl * inv, w),
            _bcast(num_x * inv, w),
            _bcast(num_y * inv, w),
            _bcast(npos_tr, w),
            _bcast(npos_tcl, w),
            _bcast(ntm, w),
            jnp.zeros((1, w), jnp.float32),
        ], axis=0)

        nneg_tr = ntm - npos_tr
        nneg_tcl = ntm - npos_tcl
        need_rare = (3.0 * npos_tr < nneg_tr) | (3.0 * npos_tcl < nneg_tcl)

        @pl.when(need_rare)
        def _():
            B = p_hbm.shape[0]
            H = p_hbm.shape[2]
            n_h = H // _CB
            n_chunks = B * n_h
            k_tr = jnp.minimum(nneg_tr, 3.0 * npos_tr)
            k_tcl = jnp.minimum(nneg_tcl, 3.0 * npos_tcl)
            ks = (k_tr, k_tcl, k_tr, k_tr)

            def dma_chunk(tc):
                b = tc // n_h
                h0 = (tc % n_h) * _CB
                cps = []
                for c in range(6):
                    cps.append(pltpu.make_async_copy(
                        p_hbm.at[b, c, pl.ds(h0, _CB), :], pbuf.at[c], sem))
                cps.append(pltpu.make_async_copy(
                    tr_hbm.at[b, pl.ds(h0, _CB), :], trbuf, sem))
                cps.append(pltpu.make_async_copy(
                    tcl_hbm.at[b, pl.ds(h0, _CB), :], tclbuf, sem))
                cps.append(pltpu.make_async_copy(
                    tm_hbm.at[b, pl.ds(h0, _CB), :], tmbuf, sem))
                cps.append(pltpu.make_async_copy(
                    x_hbm.at[b, pl.ds(h0, _CB), :], xbuf, sem))
                cps.append(pltpu.make_async_copy(
                    y_hbm.at[b, pl.ds(h0, _CB), :], ybuf, sem))
                for cp in cps:
                    cp.start()
                for cp in cps:
                    cp.wait()

            def chunk_fields():
                ctr = trbuf[...]
                ctcl = tclbuf[...]
                ctm = tmbuf[...]
                cpos_tr = (ctr * ctm) > 0
                cneg_tr = ((1 - ctr) * ctm) > 0
                cpos_tcl = (ctcl * ctm) > 0
                cneg_tcl = ((1 - ctcl) * ctm) > 0
                cce_tr = _ce2(pbuf[0], pbuf[1], ctr)
                cce_tcl = _ce2(pbuf[2], pbuf[3], ctcl)
                cl1_x = _smooth_l1(pbuf[4], xbuf[...])
                cl1_y = _smooth_l1(pbuf[5], ybuf[...])
                return ((cce_tr, cneg_tr, cpos_tr),
                        (cce_tcl, cneg_tcl, cpos_tcl),
                        (cl1_x, cneg_tr, cpos_tr),
                        (cl1_y, cneg_tr, cpos_tr))

            def probe(thrs):
                def cbody(tc, carry):
                    dma_chunk(tc)
                    fields = chunk_fields()
                    outs = []
                    for i in range(4):
                        lossv, neg, _ = fields[i]
                        bits = lax.bitcast_convert_type(lossv, jnp.int32)
                        sel = neg & (bits >= thrs[i])
                        outs.append(carry[i] + jnp.sum(sel.astype(jnp.float32)))
                    return tuple(outs)
                return lax.fori_loop(0, n_chunks, cbody, (0.0,) * 4)

            def bit_body(i, prefix):
                shift = jnp.int32(30) - i
                cands = tuple(p | (jnp.int32(1) << shift) for p in prefix)
                ge = probe(cands)
                return tuple(
                    jnp.where(ge[j] >= ks[j], cands[j], prefix[j])
                    for j in range(4))

            prefix = lax.fori_loop(0, 31, bit_body, (jnp.int32(0),) * 4)

            def fbody(tc, carry):
                dma_chunk(tc)
                fields = chunk_fields()
                outs = list(carry)
                for i in range(4):
                    lossv, neg, pos = fields[i]
                    bits = lax.bitcast_convert_type(lossv, jnp.int32)
                    gt = neg & (bits > prefix[i])
                    outs[3 * i] += jnp.sum(gt.astype(jnp.float32))
                    outs[3 * i + 1] += jnp.sum(jnp.where(gt, lossv, 0.0))
                    outs[3 * i + 2] += jnp.sum(jnp.where(pos, lossv, 0.0))
                return tuple(outs)

            fin = lax.fori_loop(0, n_chunks, fbody, (0.0,) * 12)

            dens = (npos_tr + k_tr, npos_tcl + k_tcl,
                    npos_tr + k_tr, npos_tr + k_tr)
            for i in range(4):
                gt_cnt, gt_sum, ps = fin[3 * i], fin[3 * i + 1], fin[3 * i + 2]
                kth = lax.bitcast_convert_type(prefix[i], jnp.float32)
                top = gt_sum + (ks[i] - gt_cnt) * kth
                top = jnp.where(ks[i] == 0.0, 0.0, top)
                out_ref[i:i + 1, :] = _bcast((ps + top) / dens[i], w)


def kernel(predict, tr_mask, tcl_mask, sin_map, cos_map, radii_map,
           train_mask, x_mask, y_mask):
    del sin_map, cos_map, radii_map
    B, C, H, W = predict.shape
    Hb = 128 if H % 128 == 0 else H
    grid = (B, H // Hb)

    tr_i = tr_mask.astype(jnp.int32)
    tcl_i = tcl_mask.astype(jnp.int32)
    tm_i = train_mask.astype(jnp.int32)

    p_spec = pl.BlockSpec((1, 6, Hb, W), lambda b, h: (b, 0, h, 0))
    m_spec = pl.BlockSpec((1, Hb, W), lambda b, h: (b, h, 0))
    hbm = pl.BlockSpec(memory_space=pltpu.MemorySpace.HBM)

    out = pl.pallas_call(
        _body,
        grid=grid,
        in_specs=[p_spec, m_spec, m_spec, m_spec, m_spec, m_spec,
                  hbm, hbm, hbm, hbm, hbm, hbm],
        out_specs=pl.BlockSpec((8, 128), lambda b, h: (0, 0)),
        out_shape=jax.ShapeDtypeStruct((8, 128), jnp.float32),
        scratch_shapes=[
            pltpu.VMEM((8, W), jnp.float32),
            pltpu.VMEM((6, _CB, W), jnp.float32),
            pltpu.VMEM((_CB, W), jnp.int32),
            pltpu.VMEM((_CB, W), jnp.int32),
            pltpu.VMEM((_CB, W), jnp.int32),
            pltpu.VMEM((_CB, W), jnp.float32),
            pltpu.VMEM((_CB, W), jnp.float32),
            pltpu.SemaphoreType.DMA,
        ],
    )(predict, tr_i, tcl_i, tm_i, x_mask, y_mask,
      predict, tr_i, tcl_i, tm_i, x_mask, y_mask)

    return (out[0, 0], out[1, 0], out[2, 0] / 2, out[3, 0] / 2)


# speedup vs baseline: 1.2873x; 1.2149x over previous
import jax
import jax.numpy as jnp
from jax import lax
from jax.experimental import pallas as pl
from jax.experimental.pallas import tpu as pltpu

_CB = 128


def _ce2(l0, l1, label):
    m = jnp.maximum(l0, l1)
    lse = m + jnp.log1p(jnp.exp(-jnp.abs(l0 - l1)))
    picked = jnp.where(label > 0, l1, l0)
    return lse - picked


def _smooth_l1(pred, target):
    d = jnp.abs(pred - target)
    return jnp.where(d < 1.0, 0.5 * d * d, d - 0.5)


def _colsum(x):
    return jnp.sum(x, axis=0, keepdims=True)


def _row(acc_ref, i):
    return jnp.sum(acc_ref[i:i + 1, :])


def _bcast(v, w):
    return jnp.zeros((1, w), jnp.float32) + v


def _body(p_ref, trm_ref, tclm_ref, tram_ref, xm_ref, ym_ref,
          p_hbm, tr_hbm, tcl_hbm, tm_hbm, x_hbm, y_hbm,
          out_ref, acc_ref, pbuf, trbuf, tclbuf, tmbuf, xbuf, ybuf, sem):
    first = (pl.program_id(0) == 0) & (pl.program_id(1) == 0)
    last = ((pl.program_id(0) == pl.num_programs(0) - 1)
            & (pl.program_id(1) == pl.num_programs(1) - 1))

    @pl.when(first)
    def _():
        acc_ref[...] = jnp.zeros_like(acc_ref)

    f32 = jnp.float32
    Hb = trm_ref.shape[1]
    W = trm_ref.shape[2]
    SUB = 8
    accs = [jnp.zeros((SUB, W), f32) for _ in range(7)]
    for i in range(Hb // SUB):
        sl = pl.ds(i * SUB, SUB)
        tr = trm_ref[0, sl, :]
        tcl = tclm_ref[0, sl, :]
        tm = tram_ref[0, sl, :]
        tm_f = tm.astype(f32)
        tr_f = tr.astype(f32)
        tcl_f = tcl.astype(f32)
        ce_tr = _ce2(p_ref[0, 0, sl, :], p_ref[0, 1, sl, :], tr)
        ce_tcl = _ce2(p_ref[0, 2, sl, :], p_ref[0, 3, sl, :], tcl)
        l1_x = _smooth_l1(p_ref[0, 4, sl, :], xm_ref[0, sl, :])
        l1_y = _smooth_l1(p_ref[0, 5, sl, :], ym_ref[0, sl, :])
        accs[0] += tm_f
        accs[1] += tr_f * tm_f
        accs[2] += tm_f * ce_tr
        accs[3] += tcl_f * tm_f
        accs[4] += tm_f * ce_tcl
        accs[5] += tm_f * l1_x
        accs[6] += tm_f * l1_y

    rows = [_colsum(a) for a in accs] + [jnp.zeros((1, W), f32)]
    acc_ref[...] += jnp.concatenate(rows, axis=0)

    @pl.when(last)
    def _():
        ntm = _row(acc_ref, 0)
        npos_tr = _row(acc_ref, 1)
        num_tr = _row(acc_ref, 2)
        npos_tcl = _row(acc_ref, 3)
        num_tcl = _row(acc_ref, 4)
        num_x = _row(acc_ref, 5)
        num_y = _row(acc_ref, 6)
        w = out_ref.shape[1]
        inv = 1.0 / ntm
        out_ref[...] = jnp.concatenate([
            _bcast(num_tr * inv, w),
            _bcast(num_tcl * inv, w),
            _bcast(num_x * inv, w),
            _bcast(num_y * inv, w),
            _bcast(npos_tr, w),
            _bcast(npos_tcl, w),
            _bcast(ntm, w),
            jnp.zeros((1, w), jnp.float32),
        ], axis=0)

        nneg_tr = ntm - npos_tr
        nneg_tcl = ntm - npos_tcl
        need_rare = (3.0 * npos_tr < nneg_tr) | (3.0 * npos_tcl < nneg_tcl)

        @pl.when(need_rare)
        def _():
            B = p_hbm.shape[0]
            H = p_hbm.shape[2]
            n_h = H // _CB
            n_chunks = B * n_h
            k_tr = jnp.minimum(nneg_tr, 3.0 * npos_tr)
            k_tcl = jnp.minimum(nneg_tcl, 3.0 * npos_tcl)
            ks = (k_tr, k_tcl, k_tr, k_tr)

            def dma_chunk(tc):
                b = tc // n_h
                h0 = (tc % n_h) * _CB
                cps = []
                for c in range(6):
                    cps.append(pltpu.make_async_copy(
                        p_hbm.at[b, c, pl.ds(h0, _CB), :], pbuf.at[c], sem))
                cps.append(pltpu.make_async_copy(
                    tr_hbm.at[b, pl.ds(h0, _CB), :], trbuf, sem))
                cps.append(pltpu.make_async_copy(
                    tcl_hbm.at[b, pl.ds(h0, _CB), :], tclbuf, sem))
                cps.append(pltpu.make_async_copy(
                    tm_hbm.at[b, pl.ds(h0, _CB), :], tmbuf, sem))
                cps.append(pltpu.make_async_copy(
                    x_hbm.at[b, pl.ds(h0, _CB), :], xbuf, sem))
                cps.append(pltpu.make_async_copy(
                    y_hbm.at[b, pl.ds(h0, _CB), :], ybuf, sem))
                for cp in cps:
                    cp.start()
                for cp in cps:
                    cp.wait()

            def chunk_fields():
                ctr = trbuf[...]
                ctcl = tclbuf[...]
                ctm = tmbuf[...]
                cpos_tr = (ctr * ctm) > 0
                cneg_tr = ((1 - ctr) * ctm) > 0
                cpos_tcl = (ctcl * ctm) > 0
                cneg_tcl = ((1 - ctcl) * ctm) > 0
                cce_tr = _ce2(pbuf[0], pbuf[1], ctr)
                cce_tcl = _ce2(pbuf[2], pbuf[3], ctcl)
                cl1_x = _smooth_l1(pbuf[4], xbuf[...])
                cl1_y = _smooth_l1(pbuf[5], ybuf[...])
                return ((cce_tr, cneg_tr, cpos_tr),
                        (cce_tcl, cneg_tcl, cpos_tcl),
                        (cl1_x, cneg_tr, cpos_tr),
                        (cl1_y, cneg_tr, cpos_tr))

            def probe(thrs):
                def cbody(tc, carry):
                    dma_chunk(tc)
                    fields = chunk_fields()
                    outs = []
                    for i in range(4):
                        lossv, neg, _ = fields[i]
                        bits = lax.bitcast_convert_type(lossv, jnp.int32)
                        sel = neg & (bits >= thrs[i])
                        outs.append(carry[i] + jnp.sum(sel.astype(jnp.float32)))
                    return tuple(outs)
                return lax.fori_loop(0, n_chunks, cbody, (0.0,) * 4)

            def bit_body(i, prefix):
                shift = jnp.int32(30) - i
                cands = tuple(p | (jnp.int32(1) << shift) for p in prefix)
                ge = probe(cands)
                return tuple(
                    jnp.where(ge[j] >= ks[j], cands[j], prefix[j])
                    for j in range(4))

            prefix = lax.fori_loop(0, 31, bit_body, (jnp.int32(0),) * 4)

            def fbody(tc, carry):
                dma_chunk(tc)
                fields = chunk_fields()
                outs = list(carry)
                for i in range(4):
                    lossv, neg, pos = fields[i]
                    bits = lax.bitcast_convert_type(lossv, jnp.int32)
                    gt = neg & (bits > prefix[i])
                    outs[3 * i] += jnp.sum(gt.astype(jnp.float32))
                    outs[3 * i + 1] += jnp.sum(jnp.where(gt, lossv, 0.0))
                    outs[3 * i + 2] += jnp.sum(jnp.where(pos, lossv, 0.0))
                return tuple(outs)

            fin = lax.fori_loop(0, n_chunks, fbody, (0.0,) * 12)

            dens = (npos_tr + k_tr, npos_tcl + k_tcl,
                    npos_tr + k_tr, npos_tr + k_tr)
            for i in range(4):
                gt_cnt, gt_sum, ps = fin[3 * i], fin[3 * i + 1], fin[3 * i + 2]
                kth = lax.bitcast_convert_type(prefix[i], jnp.float32)
                top = gt_sum + (ks[i] - gt_cnt) * kth
                top = jnp.where(ks[i] == 0.0, 0.0, top)
                out_ref[i:i + 1, :] = _bcast((ps + top) / dens[i], w)


def kernel(predict, tr_mask, tcl_mask, sin_map, cos_map, radii_map,
           train_mask, x_mask, y_mask):
    del sin_map, cos_map, radii_map
    B, C, H, W = predict.shape
    Hb = 256 if H % 256 == 0 else H
    grid = (B, H // Hb)

    tr_i = tr_mask.astype(jnp.int32)
    tcl_i = tcl_mask.astype(jnp.int32)
    tm_i = train_mask.astype(jnp.int32)

    p_spec = pl.BlockSpec((1, 6, Hb, W), lambda b, h: (b, 0, h, 0))
    m_spec = pl.BlockSpec((1, Hb, W), lambda b, h: (b, h, 0))
    hbm = pl.BlockSpec(memory_space=pltpu.MemorySpace.HBM)

    out = pl.pallas_call(
        _body,
        grid=grid,
        in_specs=[p_spec, m_spec, m_spec, m_spec, m_spec, m_spec,
                  hbm, hbm, hbm, hbm, hbm, hbm],
        out_specs=pl.BlockSpec((8, 128), lambda b, h: (0, 0)),
        out_shape=jax.ShapeDtypeStruct((8, 128), jnp.float32),
        scratch_shapes=[
            pltpu.VMEM((8, W), jnp.float32),
            pltpu.VMEM((6, _CB, W), jnp.float32),
            pltpu.VMEM((_CB, W), jnp.int32),
            pltpu.VMEM((_CB, W), jnp.int32),
            pltpu.VMEM((_CB, W), jnp.int32),
            pltpu.VMEM((_CB, W), jnp.float32),
            pltpu.VMEM((_CB, W), jnp.float32),
            pltpu.SemaphoreType.DMA,
        ],
    )(predict, tr_i, tcl_i, tm_i, x_mask, y_mask,
      predict, tr_i, tcl_i, tm_i, x_mask, y_mask)

    return (out[0, 0], out[1, 0], out[2, 0] / 2, out[3, 0] / 2)
